# Initial kernel scaffold; baseline (speedup 1.0000x reference)
import functools

import jax
import jax.numpy as jnp
from jax import lax
from jax.experimental import pallas as pl
from jax.experimental.pallas import tpu as pltpu
from jax.experimental.pallas import tpu_sc as plsc

N_NODES = 10000
N_PAD = 10240
E_TOTAL = 160000
T_STEPS = 4
F_IN = 128
H_DIM = 64
N_LAYERS = 2

N_WORKERS = 32
E_PER_W = E_TOTAL // N_WORKERS
KB = 125
NB = E_PER_W // KB
ROWS_PER_TILE = N_PAD // 16


@functools.cache
def _make_gconv(W):
    mesh = plsc.VectorSubcoreMesh(core_axis_name="c", subcore_axis_name="s")

    @functools.partial(
        pl.kernel,
        out_type=jax.ShapeDtypeStruct((2, N_PAD, W), jnp.float32),
        mesh=mesh,
        scratch_types=[
            pltpu.VMEM((NB, KB), jnp.int32),
            pltpu.VMEM((NB, KB), jnp.int32),
            pltpu.VMEM((NB, KB), jnp.float32),
            pltpu.VMEM((KB, W), jnp.float32),
            pltpu.VMEM((16, W), jnp.float32),
            pltpu.VMEM_SHARED((N_PAD, W), jnp.float32),
            pltpu.SemaphoreType.DMA,
        ],
    )
    def gconv(z_hbm, src_hbm, dst_hbm, w_hbm, out_hbm,
              src_v, dst_v, w_v, rows_v, zero_v, accum, sem):
        cid = lax.axis_index("c")
        sid = lax.axis_index("s")
        wid = cid * 16 + sid
        base = sid * ROWS_PER_TILE

        pltpu.sync_copy(src_hbm.at[wid], src_v)
        pltpu.sync_copy(dst_hbm.at[wid], dst_v)
        pltpu.sync_copy(w_hbm.at[wid], w_v)

        @pl.when(cid == 0)
        def _():
            pltpu.sync_copy(z_hbm.at[pl.ds(base, ROWS_PER_TILE)],
                            accum.at[pl.ds(base, ROWS_PER_TILE)])

        @pl.when(cid != 0)
        def _():
            for i in range(16):
                for c in range(W // 16):
                    zero_v[i, pl.ds(c * 16, 16)] = jnp.zeros((16,), jnp.float32)

            def zfill(i, carry):
                pltpu.sync_copy(zero_v, accum.at[pl.ds(base + i * 16, 16)])
                return carry

            lax.fori_loop(0, ROWS_PER_TILE // 16, zfill, 0)

        plsc.subcore_barrier()

        def block(j, carry):
            pltpu.async_copy(z_hbm.at[src_v.at[j]], rows_v, sem).wait()
            w_row = w_v.at[j]
            for k in range(KB):
                wk = plsc.load_gather(w_row, [jnp.full((16,), k, jnp.int32)])
                for c in range(W // 16):
                    sl = pl.ds(c * 16, 16)
                    rows_v[k, sl] = rows_v[k, sl] * wk
            pltpu.sync_copy(rows_v, accum.at[dst_v.at[j]], add=True)
            return carry

        lax.fori_loop(0, NB, block, 0)
        plsc.subcore_barrier()

        pltpu.sync_copy(accum.at[pl.ds(base, ROWS_PER_TILE)],
                        out_hbm.at[cid, pl.ds(base, ROWS_PER_TILE)])

    return gconv


_BM = 512


def _mm(xf, w):
    M, K = xf.shape
    Nout = w.shape[1]

    def body(x_ref, w_ref, o_ref):
        o_ref[...] = jnp.dot(x_ref[...], w_ref[...],
                             preferred_element_type=jnp.float32)

    return pl.pallas_call(
        body,
        grid=(M // _BM,),
        in_specs=[
            pl.BlockSpec((_BM, K), lambda i: (i, 0)),
            pl.BlockSpec((K, Nout), lambda i: (0, 0)),
        ],
        out_specs=pl.BlockSpec((_BM, Nout), lambda i: (i, 0)),
        out_shape=jax.ShapeDtypeStruct((M, Nout), jnp.float32),
    )(xf, w)


def _mm_add(h, w, addend):
    M, K = h.shape
    Nout = w.shape[1]

    def body(h_ref, w_ref, a_ref, o_ref):
        o_ref[...] = jnp.dot(h_ref[...], w_ref[...],
                             preferred_element_type=jnp.float32) + a_ref[...]

    return pl.pallas_call(
        body,
        grid=(M // _BM,),
        in_specs=[
            pl.BlockSpec((_BM, K), lambda i: (i, 0)),
            pl.BlockSpec((K, Nout), lambda i: (0, 0)),
            pl.BlockSpec((_BM, Nout), lambda i: (i, 0)),
        ],
        out_specs=pl.BlockSpec((_BM, Nout), lambda i: (i, 0)),
        out_shape=jax.ShapeDtypeStruct((M, Nout), jnp.float32),
    )(h, w, addend)


def _gates_and_sc(p0, p1, h, zxc, wc_h, b_ru8):
    M = h.shape[0]

    def body(p0_ref, p1_ref, h_ref, zxc_ref, w_ref, b_ref, sc_ref, u_ref):
        g = p0_ref[...] + p1_ref[...] + b_ref[0:1, :]
        r = jax.nn.sigmoid(g[:, :H_DIM])
        u = jax.nn.sigmoid(g[:, H_DIM:])
        rh = r * h_ref[...]
        sc_ref[...] = jnp.dot(rh, w_ref[...],
                              preferred_element_type=jnp.float32) + zxc_ref[...]
        u_ref[...] = u

    return pl.pallas_call(
        body,
        grid=(M // _BM,),
        in_specs=[
            pl.BlockSpec((_BM, 2 * H_DIM), lambda i: (i, 0)),
            pl.BlockSpec((_BM, 2 * H_DIM), lambda i: (i, 0)),
            pl.BlockSpec((_BM, H_DIM), lambda i: (i, 0)),
            pl.BlockSpec((_BM, H_DIM), lambda i: (i, 0)),
            pl.BlockSpec((H_DIM, H_DIM), lambda i: (0, 0)),
            pl.BlockSpec((8, 2 * H_DIM), lambda i: (0, 0)),
        ],
        out_specs=[
            pl.BlockSpec((_BM, H_DIM), lambda i: (i, 0)),
            pl.BlockSpec((_BM, H_DIM), lambda i: (i, 0)),
        ],
        out_shape=[
            jax.ShapeDtypeStruct((M, H_DIM), jnp.float32),
            jax.ShapeDtypeStruct((M, H_DIM), jnp.float32),
        ],
    )(p0, p1, h, zxc, wc_h, b_ru8)


def _h_update(pc0, pc1, u, h, b_c8):
    M = h.shape[0]

    def body(p0_ref, p1_ref, u_ref, h_ref, b_ref, o_ref):
        c = jnp.tanh(p0_ref[...] + p1_ref[...] + b_ref[0:1, :])
        u = u_ref[...]
        o_ref[...] = u * h_ref[...] + (1.0 - u) * c

    return pl.pallas_call(
        body,
        grid=(M // _BM,),
        in_specs=[
            pl.BlockSpec((_BM, H_DIM), lambda i: (i, 0)),
            pl.BlockSpec((_BM, H_DIM), lambda i: (i, 0)),
            pl.BlockSpec((_BM, H_DIM), lambda i: (i, 0)),
            pl.BlockSpec((_BM, H_DIM), lambda i: (i, 0)),
            pl.BlockSpec((8, H_DIM), lambda i: (0, 0)),
        ],
        out_specs=pl.BlockSpec((_BM, H_DIM), lambda i: (i, 0)),
        out_shape=jax.ShapeDtypeStruct((M, H_DIM), jnp.float32),
    )(pc0, pc1, u, h, b_c8)


def kernel(x, initial_hidden_state, edge_index, edge_weight, params):
    xt = jnp.transpose(x, (1, 0, 2))
    xt = jnp.pad(xt, ((0, 0), (0, N_PAD - N_NODES), (0, 0)))
    h = jnp.pad(initial_hidden_state,
                ((0, 0), (0, N_PAD - N_NODES), (0, 0)))
    src3 = edge_index[0].reshape(N_WORKERS, NB, KB).astype(jnp.int32)
    dst3 = edge_index[1].reshape(N_WORKERS, NB, KB).astype(jnp.int32)
    w3 = edge_weight.reshape(N_WORKERS, NB, KB)

    gconv128 = _make_gconv(2 * H_DIM)
    gconv64 = _make_gconv(H_DIM)

    cur = xt
    hid_finals = []
    for i in range(N_LAYERS):
        p = params['layer%d' % i]
        din = cur.shape[-1]
        wx = jnp.concatenate(
            [p['W_r'][:din], p['W_u'][:din], p['W_c'][:din]], axis=1)
        wh_ru = jnp.concatenate([p['W_r'][din:], p['W_u'][din:]], axis=1)
        wc_h = p['W_c'][din:]
        b_ru8 = jnp.tile(jnp.concatenate([p['b_r'], p['b_u']])[None, :], (8, 1))
        b_c8 = jnp.tile(p['b_c'][None, :], (8, 1))

        zx = _mm(cur.reshape(T_STEPS * N_PAD, din), wx)
        zx = zx.reshape(T_STEPS, N_PAD, 3 * H_DIM)

        hcur = h[i]
        outs = []
        for t in range(T_STEPS):
            s_ru = _mm_add(hcur, wh_ru, zx[t][:, :2 * H_DIM])
            pru = gconv128(s_ru, src3, dst3, w3)
            s_c, u = _gates_and_sc(pru[0], pru[1], hcur, zx[t][:, 2 * H_DIM:],
                                   wc_h, b_ru8)
            pc = gconv64(s_c, src3, dst3, w3)
            hcur = _h_update(pc[0], pc[1], u, hcur, b_c8)
            outs.append(hcur)
        hid_finals.append(hcur)
        cur = jnp.stack(outs, axis=0)

    hiddens = jnp.stack(hid_finals, axis=0)[:, :N_NODES, :]
    current = jnp.transpose(cur, (1, 0, 2))[:N_NODES]
    return (hiddens, current)


# speedup vs baseline: 4.1482x; 4.1482x over previous
import functools

import jax
import jax.numpy as jnp
from jax import lax
from jax.experimental import pallas as pl
from jax.experimental.pallas import tpu as pltpu
from jax.experimental.pallas import tpu_sc as plsc

N_NODES = 10000
N_PAD = 10240
E_TOTAL = 160000
T_STEPS = 4
F_IN = 128
H_DIM = 64
N_LAYERS = 2

N_WORKERS = 32
E_PER_W = E_TOTAL // N_WORKERS
KB = 125
NB = E_PER_W // KB
ROWS_PER_TILE = N_PAD // 16


@functools.cache
def _make_gconv(W):
    mesh = plsc.VectorSubcoreMesh(core_axis_name="c", subcore_axis_name="s")

    @functools.partial(
        pl.kernel,
        out_type=jax.ShapeDtypeStruct((2, N_PAD, W), jnp.float32),
        mesh=mesh,
        scratch_types=[
            pltpu.VMEM((NB, KB), jnp.int32),
            pltpu.VMEM((NB, KB), jnp.int32),
            pltpu.VMEM((NB, KB), jnp.float32),
            pltpu.VMEM((KB, W), jnp.float32),
            pltpu.VMEM((16, W), jnp.float32),
            pltpu.VMEM_SHARED((N_PAD, W), jnp.float32),
            pltpu.SemaphoreType.DMA,
        ],
    )
    def gconv(z_hbm, src_hbm, dst_hbm, w_hbm, out_hbm,
              src_v, dst_v, w_v, rows_v, zero_v, accum, sem):
        cid = lax.axis_index("c")
        sid = lax.axis_index("s")
        wid = cid * 16 + sid
        base = sid * ROWS_PER_TILE

        pltpu.sync_copy(src_hbm.at[wid], src_v)
        pltpu.sync_copy(dst_hbm.at[wid], dst_v)
        pltpu.sync_copy(w_hbm.at[wid], w_v)

        @pl.when(cid == 0)
        def _():
            pltpu.sync_copy(z_hbm.at[pl.ds(base, ROWS_PER_TILE)],
                            accum.at[pl.ds(base, ROWS_PER_TILE)])

        @pl.when(cid != 0)
        def _():
            for i in range(16):
                for c in range(W // 16):
                    zero_v[i, pl.ds(c * 16, 16)] = jnp.zeros((16,), jnp.float32)

            def zfill(i, carry):
                pltpu.sync_copy(zero_v, accum.at[pl.ds(base + i * 16, 16)])
                return carry

            lax.fori_loop(0, ROWS_PER_TILE // 16, zfill, 0)

        plsc.subcore_barrier()

        def block(j, carry):
            pltpu.async_copy(z_hbm.at[src_v.at[j]], rows_v, sem).wait()
            for k16 in range(0, KB, 16):
                nk = min(16, KB - k16)
                wvec = w_v[j, pl.ds(k16 - max(0, k16 + 16 - KB), 16)]
                for kk in range(nk):
                    lane = kk + (16 - nk)
                    wk = lax.gather(
                        wvec, jnp.full((16, 1), lane, jnp.int32),
                        lax.GatherDimensionNumbers(
                            offset_dims=(), collapsed_slice_dims=(0,),
                            start_index_map=(0,)),
                        slice_sizes=(1,),
                        mode=lax.GatherScatterMode.PROMISE_IN_BOUNDS)
                    k = k16 + kk
                    for c in range(W // 16):
                        sl = pl.ds(c * 16, 16)
                        rows_v[k, sl] = rows_v[k, sl] * wk
            pltpu.sync_copy(rows_v, accum.at[dst_v.at[j]], add=True)
            return carry

        lax.fori_loop(0, NB, block, 0)
        plsc.subcore_barrier()

        pltpu.sync_copy(accum.at[pl.ds(base, ROWS_PER_TILE)],
                        out_hbm.at[cid, pl.ds(base, ROWS_PER_TILE)])

    return gconv


_BM = 512


def _mm(xf, w):
    M, K = xf.shape
    Nout = w.shape[1]

    def body(x_ref, w_ref, o_ref):
        o_ref[...] = jnp.dot(x_ref[...], w_ref[...],
                             preferred_element_type=jnp.float32)

    return pl.pallas_call(
        body,
        grid=(M // _BM,),
        in_specs=[
            pl.BlockSpec((_BM, K), lambda i: (i, 0)),
            pl.BlockSpec((K, Nout), lambda i: (0, 0)),
        ],
        out_specs=pl.BlockSpec((_BM, Nout), lambda i: (i, 0)),
        out_shape=jax.ShapeDtypeStruct((M, Nout), jnp.float32),
    )(xf, w)


def _mm_add(h, w, addend):
    M, K = h.shape
    Nout = w.shape[1]

    def body(h_ref, w_ref, a_ref, o_ref):
        o_ref[...] = jnp.dot(h_ref[...], w_ref[...],
                             preferred_element_type=jnp.float32) + a_ref[...]

    return pl.pallas_call(
        body,
        grid=(M // _BM,),
        in_specs=[
            pl.BlockSpec((_BM, K), lambda i: (i, 0)),
            pl.BlockSpec((K, Nout), lambda i: (0, 0)),
            pl.BlockSpec((_BM, Nout), lambda i: (i, 0)),
        ],
        out_specs=pl.BlockSpec((_BM, Nout), lambda i: (i, 0)),
        out_shape=jax.ShapeDtypeStruct((M, Nout), jnp.float32),
    )(h, w, addend)


def _gates_and_sc(p0, p1, h, zxc, wc_h, b_ru8):
    M = h.shape[0]

    def body(p0_ref, p1_ref, h_ref, zxc_ref, w_ref, b_ref, sc_ref, u_ref):
        g = p0_ref[...] + p1_ref[...] + b_ref[0:1, :]
        r = jax.nn.sigmoid(g[:, :H_DIM])
        u = jax.nn.sigmoid(g[:, H_DIM:])
        rh = r * h_ref[...]
        sc = jnp.dot(rh, w_ref[...],
                     preferred_element_type=jnp.float32) + zxc_ref[...]
        sc_ref[...] = jnp.concatenate(
            [sc, jnp.zeros_like(sc)], axis=1)
        u_ref[...] = u

    return pl.pallas_call(
        body,
        grid=(M // _BM,),
        in_specs=[
            pl.BlockSpec((_BM, 2 * H_DIM), lambda i: (i, 0)),
            pl.BlockSpec((_BM, 2 * H_DIM), lambda i: (i, 0)),
            pl.BlockSpec((_BM, H_DIM), lambda i: (i, 0)),
            pl.BlockSpec((_BM, H_DIM), lambda i: (i, 0)),
            pl.BlockSpec((H_DIM, H_DIM), lambda i: (0, 0)),
            pl.BlockSpec((8, 2 * H_DIM), lambda i: (0, 0)),
        ],
        out_specs=[
            pl.BlockSpec((_BM, 2 * H_DIM), lambda i: (i, 0)),
            pl.BlockSpec((_BM, H_DIM), lambda i: (i, 0)),
        ],
        out_shape=[
            jax.ShapeDtypeStruct((M, 2 * H_DIM), jnp.float32),
            jax.ShapeDtypeStruct((M, H_DIM), jnp.float32),
        ],
    )(p0, p1, h, zxc, wc_h, b_ru8)


def _h_update(pc0, pc1, u, h, b_c8):
    M = h.shape[0]

    def body(p0_ref, p1_ref, u_ref, h_ref, b_ref, o_ref):
        c = jnp.tanh(p0_ref[:, :H_DIM] + p1_ref[:, :H_DIM] + b_ref[0:1, :])
        u = u_ref[...]
        o_ref[...] = u * h_ref[...] + (1.0 - u) * c

    return pl.pallas_call(
        body,
        grid=(M // _BM,),
        in_specs=[
            pl.BlockSpec((_BM, 2 * H_DIM), lambda i: (i, 0)),
            pl.BlockSpec((_BM, 2 * H_DIM), lambda i: (i, 0)),
            pl.BlockSpec((_BM, H_DIM), lambda i: (i, 0)),
            pl.BlockSpec((_BM, H_DIM), lambda i: (i, 0)),
            pl.BlockSpec((8, H_DIM), lambda i: (0, 0)),
        ],
        out_specs=pl.BlockSpec((_BM, H_DIM), lambda i: (i, 0)),
        out_shape=jax.ShapeDtypeStruct((M, H_DIM), jnp.float32),
    )(pc0, pc1, u, h, b_c8)


def kernel(x, initial_hidden_state, edge_index, edge_weight, params):
    xt = jnp.transpose(x, (1, 0, 2))
    xt = jnp.pad(xt, ((0, 0), (0, N_PAD - N_NODES), (0, 0)))
    h = jnp.pad(initial_hidden_state,
                ((0, 0), (0, N_PAD - N_NODES), (0, 0)))
    src3 = edge_index[0].reshape(N_WORKERS, NB, KB).astype(jnp.int32)
    dst3 = edge_index[1].reshape(N_WORKERS, NB, KB).astype(jnp.int32)
    w3 = edge_weight.reshape(N_WORKERS, NB, KB)

    gconv128 = _make_gconv(2 * H_DIM)

    cur = xt
    hid_finals = []
    for i in range(N_LAYERS):
        p = params['layer%d' % i]
        din = cur.shape[-1]
        wx = jnp.concatenate(
            [p['W_r'][:din], p['W_u'][:din], p['W_c'][:din]], axis=1)
        wh_ru = jnp.concatenate([p['W_r'][din:], p['W_u'][din:]], axis=1)
        wc_h = p['W_c'][din:]
        b_ru8 = jnp.tile(jnp.concatenate([p['b_r'], p['b_u']])[None, :], (8, 1))
        b_c8 = jnp.tile(p['b_c'][None, :], (8, 1))

        zx = _mm(cur.reshape(T_STEPS * N_PAD, din), wx)
        zx = zx.reshape(T_STEPS, N_PAD, 3 * H_DIM)

        hcur = h[i]
        outs = []
        for t in range(T_STEPS):
            s_ru = _mm_add(hcur, wh_ru, zx[t][:, :2 * H_DIM])
            pru = gconv128(s_ru, src3, dst3, w3)
            s_c, u = _gates_and_sc(pru[0], pru[1], hcur, zx[t][:, 2 * H_DIM:],
                                   wc_h, b_ru8)
            pc = gconv128(s_c, src3, dst3, w3)
            hcur = _h_update(pc[0], pc[1], u, hcur, b_c8)
            outs.append(hcur)
        hid_finals.append(hcur)
        cur = jnp.stack(outs, axis=0)

    hiddens = jnp.stack(hid_finals, axis=0)[:, :N_NODES, :]
    current = jnp.transpose(cur, (1, 0, 2))[:N_NODES]
    return (hiddens, current)


# speedup vs baseline: 5.5748x; 1.3439x over previous
import functools

import jax
import jax.numpy as jnp
from jax import lax
from jax.experimental import pallas as pl
from jax.experimental.pallas import tpu as pltpu
from jax.experimental.pallas import tpu_sc as plsc

N_NODES = 10000
N_PAD = 10240
E_TOTAL = 160000
T_STEPS = 4
F_IN = 128
H_DIM = 64
N_LAYERS = 2

N_WORKERS = 32
E_PER_W = E_TOTAL // N_WORKERS
KB = 125
NB = E_PER_W // KB
N_ACC = 10112
ROWS_PER_TILE = N_ACC // 16

_BCAST_DNUMS = lax.GatherDimensionNumbers(
    offset_dims=(), collapsed_slice_dims=(0,), start_index_map=(0,))


def _lane_bcast(vec, lane):
    return lax.gather(vec, jnp.full((16, 1), lane, jnp.int32), _BCAST_DNUMS,
                      slice_sizes=(1,),
                      mode=lax.GatherScatterMode.PROMISE_IN_BOUNDS)


@functools.cache
def _make_gconv(W):
    mesh = plsc.VectorSubcoreMesh(core_axis_name="c", subcore_axis_name="s")

    @functools.partial(
        pl.kernel,
        out_type=jax.ShapeDtypeStruct((2, N_ACC, W), jnp.float32),
        mesh=mesh,
        scratch_types=[
            pltpu.VMEM((NB, KB), jnp.int32),
            pltpu.VMEM((NB, KB), jnp.int32),
            pltpu.VMEM((NB, KB), jnp.float32),
            pltpu.VMEM((2, KB, W), jnp.float32),
            pltpu.VMEM((16, W), jnp.float32),
            pltpu.VMEM_SHARED((N_ACC, W), jnp.float32),
            pltpu.SemaphoreType.DMA,
        ],
    )
    def gconv(z_hbm, src_hbm, dst_hbm, w_hbm, out_hbm,
              src_v, dst_v, w_v, rows_v, zero_v, accum, sem):
        cid = lax.axis_index("c")
        sid = lax.axis_index("s")
        wid = cid * 16 + sid
        base = sid * ROWS_PER_TILE

        pltpu.sync_copy(src_hbm.at[wid], src_v)
        pltpu.sync_copy(dst_hbm.at[wid], dst_v)
        pltpu.sync_copy(w_hbm.at[wid], w_v)

        @pl.when(cid == 0)
        def _():
            pltpu.sync_copy(z_hbm.at[pl.ds(base, ROWS_PER_TILE)],
                            accum.at[pl.ds(base, ROWS_PER_TILE)])

        @pl.when(cid != 0)
        def _():
            for i in range(16):
                for c in range(W // 16):
                    zero_v[i, pl.ds(c * 16, 16)] = jnp.zeros((16,), jnp.float32)

            def zfill(i, carry):
                pltpu.sync_copy(zero_v, accum.at[pl.ds(base + i * 16, 16)])
                return carry

            lax.fori_loop(0, ROWS_PER_TILE // 16, zfill, 0)
            pltpu.sync_copy(zero_v.at[pl.ds(0, ROWS_PER_TILE % 16)],
                            accum.at[pl.ds(base + 16 * (ROWS_PER_TILE // 16),
                                           ROWS_PER_TILE % 16)])

        plsc.subcore_barrier()

        def block(j, carry):
            b = lax.rem(j, 2)

            @pl.when(j == 0)
            def _():
                pltpu.async_copy(z_hbm.at[src_v.at[j]], rows_v.at[b], sem)

            pltpu.make_async_copy(z_hbm.at[src_v.at[j]], rows_v.at[b],
                                  sem).wait()

            @pl.when(j + 1 < NB)
            def _():
                pltpu.async_copy(z_hbm.at[src_v.at[j + 1]], rows_v.at[1 - b],
                                 sem)
            for k16 in range(0, KB, 16):
                nk = min(16, KB - k16)
                wvec = w_v[j, pl.ds(k16 - max(0, k16 + 16 - KB), 16)]
                for kk in range(nk):
                    lane = kk + (16 - nk)
                    wk = _lane_bcast(wvec, lane)
                    k = k16 + kk
                    for c in range(W // 16):
                        sl = pl.ds(c * 16, 16)
                        rows_v[b, k, sl] = rows_v[b, k, sl] * wk
            pltpu.sync_copy(rows_v.at[b], accum.at[dst_v.at[j]], add=True)
            return carry

        lax.fori_loop(0, NB, block, 0)
        plsc.subcore_barrier()

        pltpu.sync_copy(accum.at[pl.ds(base, ROWS_PER_TILE)],
                        out_hbm.at[cid, pl.ds(base, ROWS_PER_TILE)])

    return gconv


_BM = 512


def _mm(xf, w):
    M, K = xf.shape
    Nout = w.shape[1]

    def body(x_ref, w_ref, o_ref):
        o_ref[...] = jnp.dot(x_ref[...], w_ref[...],
                             preferred_element_type=jnp.float32)

    return pl.pallas_call(
        body,
        grid=(M // _BM,),
        in_specs=[
            pl.BlockSpec((_BM, K), lambda i: (i, 0)),
            pl.BlockSpec((K, Nout), lambda i: (0, 0)),
        ],
        out_specs=pl.BlockSpec((_BM, Nout), lambda i: (i, 0)),
        out_shape=jax.ShapeDtypeStruct((M, Nout), jnp.float32),
    )(xf, w)


def _mm_add(h, w, addend):
    M, K = h.shape
    Nout = w.shape[1]

    def body(h_ref, w_ref, a_ref, o_ref):
        o_ref[...] = jnp.dot(h_ref[...], w_ref[...],
                             preferred_element_type=jnp.float32) + a_ref[...]

    return pl.pallas_call(
        body,
        grid=(M // _BM,),
        in_specs=[
            pl.BlockSpec((_BM, K), lambda i: (i, 0)),
            pl.BlockSpec((K, Nout), lambda i: (0, 0)),
            pl.BlockSpec((_BM, Nout), lambda i: (i, 0)),
        ],
        out_specs=pl.BlockSpec((_BM, Nout), lambda i: (i, 0)),
        out_shape=jax.ShapeDtypeStruct((M, Nout), jnp.float32),
    )(h, w, addend)


def _gates_and_sc(p0, p1, h, zxc, wc_h, b_ru8):
    M = h.shape[0]

    def body(p0_ref, p1_ref, h_ref, zxc_ref, w_ref, b_ref, sc_ref, u_ref):
        g = p0_ref[...] + p1_ref[...] + b_ref[0:1, :]
        r = jax.nn.sigmoid(g[:, :H_DIM])
        u = jax.nn.sigmoid(g[:, H_DIM:])
        rh = r * h_ref[...]
        sc = jnp.dot(rh, w_ref[...],
                     preferred_element_type=jnp.float32) + zxc_ref[...]
        sc_ref[...] = jnp.concatenate([sc, jnp.zeros_like(sc)], axis=1)
        u_ref[...] = u

    return pl.pallas_call(
        body,
        grid=(M // _BM,),
        in_specs=[
            pl.BlockSpec((_BM, 2 * H_DIM), lambda i: (i, 0)),
            pl.BlockSpec((_BM, 2 * H_DIM), lambda i: (i, 0)),
            pl.BlockSpec((_BM, H_DIM), lambda i: (i, 0)),
            pl.BlockSpec((_BM, H_DIM), lambda i: (i, 0)),
            pl.BlockSpec((H_DIM, H_DIM), lambda i: (0, 0)),
            pl.BlockSpec((8, 2 * H_DIM), lambda i: (0, 0)),
        ],
        out_specs=[
            pl.BlockSpec((_BM, 2 * H_DIM), lambda i: (i, 0)),
            pl.BlockSpec((_BM, H_DIM), lambda i: (i, 0)),
        ],
        out_shape=[
            jax.ShapeDtypeStruct((M, 2 * H_DIM), jnp.float32),
            jax.ShapeDtypeStruct((M, H_DIM), jnp.float32),
        ],
    )(p0, p1, h, zxc, wc_h, b_ru8)


def _h_update(pc0, pc1, u, h, b_c8):
    M = h.shape[0]

    def body(p0_ref, p1_ref, u_ref, h_ref, b_ref, o_ref):
        c = jnp.tanh(p0_ref[:, :H_DIM] + p1_ref[:, :H_DIM] + b_ref[0:1, :])
        u = u_ref[...]
        o_ref[...] = u * h_ref[...] + (1.0 - u) * c

    return pl.pallas_call(
        body,
        grid=(M // _BM,),
        in_specs=[
            pl.BlockSpec((_BM, 2 * H_DIM), lambda i: (i, 0)),
            pl.BlockSpec((_BM, 2 * H_DIM), lambda i: (i, 0)),
            pl.BlockSpec((_BM, H_DIM), lambda i: (i, 0)),
            pl.BlockSpec((_BM, H_DIM), lambda i: (i, 0)),
            pl.BlockSpec((8, H_DIM), lambda i: (0, 0)),
        ],
        out_specs=pl.BlockSpec((_BM, H_DIM), lambda i: (i, 0)),
        out_shape=jax.ShapeDtypeStruct((M, H_DIM), jnp.float32),
    )(pc0, pc1, u, h, b_c8)


def kernel(x, initial_hidden_state, edge_index, edge_weight, params):
    xt = jnp.transpose(x, (1, 0, 2))
    xt = jnp.pad(xt, ((0, 0), (0, N_PAD - N_NODES), (0, 0)))
    h = jnp.pad(initial_hidden_state,
                ((0, 0), (0, N_PAD - N_NODES), (0, 0)))
    src3 = edge_index[0].astype(jnp.int32).reshape(N_WORKERS, NB, KB)
    dst3 = edge_index[1].astype(jnp.int32).reshape(N_WORKERS, NB, KB)
    w3 = edge_weight.reshape(N_WORKERS, NB, KB)

    gconv128 = _make_gconv(2 * H_DIM)

    cur = xt
    hid_finals = []
    for i in range(N_LAYERS):
        p = params['layer%d' % i]
        din = cur.shape[-1]
        wx = jnp.concatenate(
            [p['W_r'][:din], p['W_u'][:din], p['W_c'][:din]], axis=1)
        wh_ru = jnp.concatenate([p['W_r'][din:], p['W_u'][din:]], axis=1)
        wc_h = p['W_c'][din:]
        b_ru8 = jnp.tile(jnp.concatenate([p['b_r'], p['b_u']])[None, :], (8, 1))
        b_c8 = jnp.tile(p['b_c'][None, :], (8, 1))

        zx = _mm(cur.reshape(T_STEPS * N_PAD, din), wx)
        zx = zx.reshape(T_STEPS, N_PAD, 3 * H_DIM)

        hcur = h[i]
        outs = []
        for t in range(T_STEPS):
            s_ru = _mm_add(hcur, wh_ru, zx[t][:, :2 * H_DIM])
            pru = gconv128(s_ru, src3, dst3, w3)
            s_c, u = _gates_and_sc(pru[0], pru[1], hcur, zx[t][:, 2 * H_DIM:],
                                   wc_h, b_ru8)
            pc = gconv128(s_c, src3, dst3, w3)
            hcur = _h_update(pc[0], pc[1], u, hcur, b_c8)
            outs.append(hcur)
        hid_finals.append(hcur)
        cur = jnp.stack(outs, axis=0)

    hiddens = jnp.stack(hid_finals, axis=0)[:, :N_NODES, :]
    current = jnp.transpose(cur, (1, 0, 2))[:N_NODES]
    return (hiddens, current)


# speedup vs baseline: 5.7650x; 1.0341x over previous
import functools

import jax
import jax.numpy as jnp
from jax import lax
from jax.experimental import pallas as pl
from jax.experimental.pallas import tpu as pltpu
from jax.experimental.pallas import tpu_sc as plsc

N_NODES = 10000
N_PAD = 10240
E_TOTAL = 160000
T_STEPS = 4
F_IN = 128
H_DIM = 64
N_LAYERS = 2

N_WORKERS = 32
E_PER_W = E_TOTAL // N_WORKERS
KB = 125
NB = E_PER_W // KB
N_ACC = 10112
ROWS_PER_TILE = N_ACC // 16

_BCAST_DNUMS = lax.GatherDimensionNumbers(
    offset_dims=(), collapsed_slice_dims=(0,), start_index_map=(0,))


def _lane_bcast(vec, lane):
    return lax.gather(vec, jnp.full((16, 1), lane, jnp.int32), _BCAST_DNUMS,
                      slice_sizes=(1,),
                      mode=lax.GatherScatterMode.PROMISE_IN_BOUNDS)


@functools.cache
def _make_gconv(W):
    mesh = plsc.VectorSubcoreMesh(core_axis_name="c", subcore_axis_name="s")

    @functools.partial(
        pl.kernel,
        out_type=jax.ShapeDtypeStruct((2, N_ACC, W), jnp.float32),
        mesh=mesh,
        scratch_types=[
            pltpu.VMEM((NB, KB), jnp.int32),
            pltpu.VMEM((NB, KB), jnp.int32),
            pltpu.VMEM((NB, KB), jnp.float32),
            pltpu.VMEM((2, KB, W), jnp.float32),
            pltpu.VMEM((16, W), jnp.float32),
            pltpu.VMEM_SHARED((N_ACC, W), jnp.float32),
            pltpu.SemaphoreType.DMA,
        ],
    )
    def gconv(z_hbm, src_hbm, dst_hbm, w_hbm, out_hbm,
              src_v, dst_v, w_v, rows_v, zero_v, accum, sem):
        cid = lax.axis_index("c")
        sid = lax.axis_index("s")
        wid = cid * 16 + sid
        base = sid * ROWS_PER_TILE

        pltpu.sync_copy(src_hbm.at[wid], src_v)
        pltpu.sync_copy(dst_hbm.at[wid], dst_v)
        pltpu.sync_copy(w_hbm.at[wid], w_v)

        @pl.when(cid == 0)
        def _():
            pltpu.sync_copy(z_hbm.at[pl.ds(base, ROWS_PER_TILE)],
                            accum.at[pl.ds(base, ROWS_PER_TILE)])

        @pl.when(cid != 0)
        def _():
            for i in range(16):
                for c in range(W // 16):
                    zero_v[i, pl.ds(c * 16, 16)] = jnp.zeros((16,), jnp.float32)

            def zfill(i, carry):
                pltpu.sync_copy(zero_v, accum.at[pl.ds(base + i * 16, 16)])
                return carry

            lax.fori_loop(0, ROWS_PER_TILE // 16, zfill, 0)
            pltpu.sync_copy(zero_v.at[pl.ds(0, ROWS_PER_TILE % 16)],
                            accum.at[pl.ds(base + 16 * (ROWS_PER_TILE // 16),
                                           ROWS_PER_TILE % 16)])

        plsc.subcore_barrier()

        def block(j, carry):
            b = lax.rem(j, 2)

            @pl.when(j == 0)
            def _():
                pltpu.async_copy(z_hbm.at[src_v.at[j]], rows_v.at[b], sem)

            pltpu.make_async_copy(z_hbm.at[src_v.at[j]], rows_v.at[b],
                                  sem).wait()

            @pl.when(j + 1 < NB)
            def _():
                pltpu.async_copy(z_hbm.at[src_v.at[j + 1]], rows_v.at[1 - b],
                                 sem)
            for k16 in range(0, KB, 16):
                nk = min(16, KB - k16)
                wvec = w_v[j, pl.ds(k16 - max(0, k16 + 16 - KB), 16)]
                for kk in range(nk):
                    lane = kk + (16 - nk)
                    wk = _lane_bcast(wvec, lane)
                    k = k16 + kk
                    for c in range(W // 16):
                        sl = pl.ds(c * 16, 16)
                        rows_v[b, k, sl] = rows_v[b, k, sl] * wk
            pltpu.sync_copy(rows_v.at[b], accum.at[dst_v.at[j]], add=True)
            return carry

        lax.fori_loop(0, NB, block, 0)
        plsc.subcore_barrier()

        pltpu.sync_copy(accum.at[pl.ds(base, ROWS_PER_TILE)],
                        out_hbm.at[cid, pl.ds(base, ROWS_PER_TILE)])

    return gconv


_BM = 512


def _mm(xf, w):
    M, K = xf.shape
    Nout = w.shape[1]

    def body(x_ref, w_ref, o_ref):
        o_ref[...] = jnp.dot(x_ref[...], w_ref[...],
                             preferred_element_type=jnp.float32)

    return pl.pallas_call(
        body,
        grid=(M // _BM,),
        in_specs=[
            pl.BlockSpec((_BM, K), lambda i: (i, 0)),
            pl.BlockSpec((K, Nout), lambda i: (0, 0)),
        ],
        out_specs=pl.BlockSpec((_BM, Nout), lambda i: (i, 0)),
        out_shape=jax.ShapeDtypeStruct((M, Nout), jnp.float32),
    )(xf, w)


def _mm_add(h, w, addend):
    M, K = h.shape
    Nout = w.shape[1]

    def body(h_ref, w_ref, a_ref, o_ref):
        o_ref[...] = jnp.dot(h_ref[...], w_ref[...],
                             preferred_element_type=jnp.float32) + a_ref[...]

    return pl.pallas_call(
        body,
        grid=(M // _BM,),
        in_specs=[
            pl.BlockSpec((_BM, K), lambda i: (i, 0)),
            pl.BlockSpec((K, Nout), lambda i: (0, 0)),
            pl.BlockSpec((_BM, Nout), lambda i: (i, 0)),
        ],
        out_specs=pl.BlockSpec((_BM, Nout), lambda i: (i, 0)),
        out_shape=jax.ShapeDtypeStruct((M, Nout), jnp.float32),
    )(h, w, addend)


def _gates_and_sc(p0, p1, h, zxc, wc_h, b_ru8):
    M = h.shape[0]

    def body(p0_ref, p1_ref, h_ref, zxc_ref, w_ref, b_ref, sc_ref, u_ref):
        g = p0_ref[...] + p1_ref[...] + b_ref[0:1, :]
        r = jax.nn.sigmoid(g[:, :H_DIM])
        u = jax.nn.sigmoid(g[:, H_DIM:])
        rh = r * h_ref[...]
        sc = jnp.dot(rh, w_ref[...],
                     preferred_element_type=jnp.float32) + zxc_ref[...]
        sc_ref[...] = jnp.concatenate([sc, jnp.zeros_like(sc)], axis=1)
        u_ref[...] = u

    return pl.pallas_call(
        body,
        grid=(M // _BM,),
        in_specs=[
            pl.BlockSpec((_BM, 2 * H_DIM), lambda i: (i, 0)),
            pl.BlockSpec((_BM, 2 * H_DIM), lambda i: (i, 0)),
            pl.BlockSpec((_BM, H_DIM), lambda i: (i, 0)),
            pl.BlockSpec((_BM, H_DIM), lambda i: (i, 0)),
            pl.BlockSpec((H_DIM, H_DIM), lambda i: (0, 0)),
            pl.BlockSpec((8, 2 * H_DIM), lambda i: (0, 0)),
        ],
        out_specs=[
            pl.BlockSpec((_BM, 2 * H_DIM), lambda i: (i, 0)),
            pl.BlockSpec((_BM, H_DIM), lambda i: (i, 0)),
        ],
        out_shape=[
            jax.ShapeDtypeStruct((M, 2 * H_DIM), jnp.float32),
            jax.ShapeDtypeStruct((M, H_DIM), jnp.float32),
        ],
    )(p0, p1, h, zxc, wc_h, b_ru8)


def _h_update_mm(pc0, pc1, u, h, b_c8, wh_ru, zxru_next):
    M = h.shape[0]

    def body(p0_ref, p1_ref, u_ref, h_ref, b_ref, w_ref, zx_ref,
             h_out_ref, s_ref):
        c = jnp.tanh(p0_ref[:, :H_DIM] + p1_ref[:, :H_DIM] + b_ref[0:1, :])
        u = u_ref[...]
        hn = u * h_ref[...] + (1.0 - u) * c
        h_out_ref[...] = hn
        s_ref[...] = jnp.dot(hn, w_ref[...],
                             preferred_element_type=jnp.float32) + zx_ref[...]

    return pl.pallas_call(
        body,
        grid=(M // _BM,),
        in_specs=[
            pl.BlockSpec((_BM, 2 * H_DIM), lambda i: (i, 0)),
            pl.BlockSpec((_BM, 2 * H_DIM), lambda i: (i, 0)),
            pl.BlockSpec((_BM, H_DIM), lambda i: (i, 0)),
            pl.BlockSpec((_BM, H_DIM), lambda i: (i, 0)),
            pl.BlockSpec((8, H_DIM), lambda i: (0, 0)),
            pl.BlockSpec((H_DIM, 2 * H_DIM), lambda i: (0, 0)),
            pl.BlockSpec((_BM, 2 * H_DIM), lambda i: (i, 0)),
        ],
        out_specs=[
            pl.BlockSpec((_BM, H_DIM), lambda i: (i, 0)),
            pl.BlockSpec((_BM, 2 * H_DIM), lambda i: (i, 0)),
        ],
        out_shape=[
            jax.ShapeDtypeStruct((M, H_DIM), jnp.float32),
            jax.ShapeDtypeStruct((M, 2 * H_DIM), jnp.float32),
        ],
    )(pc0, pc1, u, h, b_c8, wh_ru, zxru_next)


def _h_update(pc0, pc1, u, h, b_c8):
    M = h.shape[0]

    def body(p0_ref, p1_ref, u_ref, h_ref, b_ref, o_ref):
        c = jnp.tanh(p0_ref[:, :H_DIM] + p1_ref[:, :H_DIM] + b_ref[0:1, :])
        u = u_ref[...]
        o_ref[...] = u * h_ref[...] + (1.0 - u) * c

    return pl.pallas_call(
        body,
        grid=(M // _BM,),
        in_specs=[
            pl.BlockSpec((_BM, 2 * H_DIM), lambda i: (i, 0)),
            pl.BlockSpec((_BM, 2 * H_DIM), lambda i: (i, 0)),
            pl.BlockSpec((_BM, H_DIM), lambda i: (i, 0)),
            pl.BlockSpec((_BM, H_DIM), lambda i: (i, 0)),
            pl.BlockSpec((8, H_DIM), lambda i: (0, 0)),
        ],
        out_specs=pl.BlockSpec((_BM, H_DIM), lambda i: (i, 0)),
        out_shape=jax.ShapeDtypeStruct((M, H_DIM), jnp.float32),
    )(pc0, pc1, u, h, b_c8)


def kernel(x, initial_hidden_state, edge_index, edge_weight, params):
    xt = jnp.transpose(x, (1, 0, 2))
    xt = jnp.pad(xt, ((0, 0), (0, N_PAD - N_NODES), (0, 0)))
    h = jnp.pad(initial_hidden_state,
                ((0, 0), (0, N_PAD - N_NODES), (0, 0)))
    src3 = edge_index[0].astype(jnp.int32).reshape(N_WORKERS, NB, KB)
    dst3 = edge_index[1].astype(jnp.int32).reshape(N_WORKERS, NB, KB)
    w3 = edge_weight.reshape(N_WORKERS, NB, KB)

    gconv128 = _make_gconv(2 * H_DIM)

    cur = xt
    hid_finals = []
    for i in range(N_LAYERS):
        p = params['layer%d' % i]
        din = cur.shape[-1]
        wx = jnp.concatenate(
            [p['W_r'][:din], p['W_u'][:din], p['W_c'][:din]], axis=1)
        wh_ru = jnp.concatenate([p['W_r'][din:], p['W_u'][din:]], axis=1)
        wc_h = p['W_c'][din:]
        b_ru8 = jnp.tile(jnp.concatenate([p['b_r'], p['b_u']])[None, :], (8, 1))
        b_c8 = jnp.tile(p['b_c'][None, :], (8, 1))

        zx = _mm(cur.reshape(T_STEPS * N_PAD, din), wx)
        zx = zx.reshape(T_STEPS, N_PAD, 3 * H_DIM)

        hcur = h[i]
        outs = []
        s_ru = _mm_add(hcur, wh_ru, zx[0][:, :2 * H_DIM])
        for t in range(T_STEPS):
            pru = gconv128(s_ru, src3, dst3, w3)
            s_c, u = _gates_and_sc(pru[0], pru[1], hcur, zx[t][:, 2 * H_DIM:],
                                   wc_h, b_ru8)
            pc = gconv128(s_c, src3, dst3, w3)
            if t + 1 < T_STEPS:
                hcur, s_ru = _h_update_mm(pc[0], pc[1], u, hcur, b_c8,
                                          wh_ru, zx[t + 1][:, :2 * H_DIM])
            else:
                hcur = _h_update(pc[0], pc[1], u, hcur, b_c8)
            outs.append(hcur)
        hid_finals.append(hcur)
        cur = jnp.stack(outs, axis=0)

    hiddens = jnp.stack(hid_finals, axis=0)[:, :N_NODES, :]
    current = jnp.transpose(cur, (1, 0, 2))[:N_NODES]
    return (hiddens, current)
